# baseline (speedup 1.0000x reference)
import functools

import jax
import jax.numpy as jnp
import numpy as np
from jax import lax
from jax.experimental import pallas as pl
from jax.experimental.pallas import tpu as pltpu
from jax.experimental.pallas import tpu_sc as plsc

B = 4096
N = 128
NP = N + 1
D = N * (N + 1) // 2
NCHUNK = D // 16
UNROLL = 8

_NUM_CORES = 2
_NUM_SUBCORES = 16
_NUM_WORKERS = _NUM_CORES * _NUM_SUBCORES
ROWS_PER_WORKER = B // _NUM_WORKERS


def _scatter_table() -> np.ndarray:
    iu, ju = np.triu_indices(N)
    return (iu * 256 + ju).astype(np.int32)


_mesh = plsc.VectorSubcoreMesh(core_axis_name="c", subcore_axis_name="s")


@functools.partial(
    pl.kernel,
    out_type=jax.ShapeDtypeStruct((B * N, N), jnp.float32),
    mesh=_mesh,
    compiler_params=pltpu.CompilerParams(needs_layout_passes=False,
                                         use_tc_tiling_on_sc=True),
    scratch_types=[
        pltpu.VMEM((D,), jnp.int32),
        pltpu.VMEM((1, D), jnp.float32),
        pltpu.VMEM((1, D), jnp.float32),
        pltpu.VMEM((N, NP), jnp.float32),
        pltpu.VMEM((N, NP), jnp.float32),
        pltpu.SemaphoreType.DMA,
        pltpu.SemaphoreType.DMA,
        pltpu.SemaphoreType.DMA,
        pltpu.SemaphoreType.DMA,
    ],
)
def _unvec_kernel(x_hbm, idx_hbm, out_hbm,
                  idx_v, in_a, in_b, out_a, out_b,
                  sin_a, sin_b, sout_a, sout_b):
    wid = lax.axis_index("s") * _NUM_CORES + lax.axis_index("c")
    base = wid * ROWS_PER_WORKER
    pltpu.sync_copy(idx_hbm, idx_v)

    in_bufs = (in_a, in_b)
    out_bufs = (out_a, out_b)
    in_sems = (sin_a, sin_b)
    out_sems = (sout_a, sout_b)

    def start_in(g, s):
        b = base + g
        pltpu.async_copy(x_hbm.at[pl.ds(b, 1), :], in_bufs[s], in_sems[s])

    def wait_in(s):
        pltpu.make_async_copy(
            x_hbm.at[pl.ds(base, 1), :], in_bufs[s], in_sems[s]).wait()

    def start_out(g, s):
        b = base + g
        pltpu.async_copy(out_bufs[s].at[:, pl.ds(0, N)],
                         out_hbm.at[pl.ds(b * N, N)], out_sems[s])

    def wait_out(s):
        pltpu.make_async_copy(
            out_bufs[s].at[:, pl.ds(0, N)],
            out_hbm.at[pl.ds(base * N, N)], out_sems[s]).wait()

    start_in(0, 0)

    def outer(g2, carry):
        for s in range(2):
            g = g2 * 2 + s
            wait_in(s)

            @pl.when(g + 1 < ROWS_PER_WORKER)
            def _():
                start_in(g + 1, 1 - s)

            @pl.when(g >= 2)
            def _():
                wait_out(s)

            src = in_bufs[s]
            dst = out_bufs[s]

            @plsc.parallel_loop(0, NCHUNK, 1, unroll=UNROLL)
            def chunk(k, src=src, dst=dst):
                off = k * 16
                p = idx_v[pl.ds(off, 16)]
                a = lax.shift_right_logical(p, 8)
                b2 = lax.bitwise_and(p, 255)
                v = src[0, pl.ds(off, 16)]
                plsc.store_scatter(dst, [a, b2], v)
                plsc.store_scatter(dst, [b2, a], v)

            start_out(g, s)
        return carry

    lax.fori_loop(0, ROWS_PER_WORKER // 2, outer, 0, unroll=False)
    wait_out(0)
    wait_out(1)


def kernel(input):
    idx = jnp.asarray(_scatter_table())
    out = _unvec_kernel(input, idx)
    return out.reshape(B, N, N)


# speedup vs baseline: 1.7151x; 1.7151x over previous
import functools

import jax
import jax.numpy as jnp
import numpy as np
from jax import lax
from jax.experimental import pallas as pl
from jax.experimental.pallas import tpu as pltpu
from jax.experimental.pallas import tpu_sc as plsc

B = 4096
N = 128
NP = N + 1
D = N * (N + 1) // 2
NN = N * N
NNP = N * NP
NCHUNK = D // 16
UNROLL = 8

_NUM_CORES = 2
_NUM_SUBCORES = 16
_NUM_WORKERS = _NUM_CORES * _NUM_SUBCORES
ROWS_PER_WORKER = B // _NUM_WORKERS
CB = 2
NBLK = ROWS_PER_WORKER // CB


def _scatter_table() -> np.ndarray:
    iu, ju = np.triu_indices(N)
    return (iu * 256 + ju).astype(np.int32)


_mesh = plsc.VectorSubcoreMesh(core_axis_name="c", subcore_axis_name="s")


@functools.partial(
    pl.kernel,
    out_type=jax.ShapeDtypeStruct((B * N, N), jnp.float32),
    mesh=_mesh,
    compiler_params=pltpu.CompilerParams(needs_layout_passes=False, use_tc_tiling_on_sc=False),
    scratch_types=[
        pltpu.VMEM((D,), jnp.int32),
        pltpu.VMEM((CB, D), jnp.float32),
        pltpu.VMEM((CB, D), jnp.float32),
        pltpu.VMEM((CB * N, NP), jnp.float32),
        pltpu.VMEM((CB * N, NP), jnp.float32),
        pltpu.SemaphoreType.DMA,
        pltpu.SemaphoreType.DMA,
        pltpu.SemaphoreType.DMA,
        pltpu.SemaphoreType.DMA,
    ],
)
def _unvec_kernel(x_hbm, idx_hbm, out_hbm,
                  idx_v, in_a, in_b, out_a, out_b,
                  sin_a, sin_b, sout_a, sout_b):
    wid = lax.axis_index("s") * _NUM_CORES + lax.axis_index("c")
    base = wid * ROWS_PER_WORKER
    pltpu.sync_copy(idx_hbm, idx_v)

    in_bufs = (in_a, in_b)
    out_bufs = (out_a, out_b)
    in_sems = (sin_a, sin_b)
    out_sems = (sout_a, sout_b)

    def start_in(g, s):
        b = base + g * CB
        pltpu.async_copy(x_hbm.at[pl.ds(b, CB), :], in_bufs[s], in_sems[s])

    def wait_in(s):
        pltpu.make_async_copy(
            x_hbm.at[pl.ds(base, CB), :], in_bufs[s], in_sems[s]).wait()

    def start_out(g, s):
        b = base + g * CB
        pltpu.async_copy(out_bufs[s].at[:, pl.ds(0, N)],
                         out_hbm.at[pl.ds(b * N, CB * N)], out_sems[s])

    def wait_out(s):
        pltpu.make_async_copy(
            out_bufs[s].at[:, pl.ds(0, N)],
            out_hbm.at[pl.ds(base * N, CB * N)], out_sems[s]).wait()

    start_in(0, 0)

    def outer(g2, carry):
        for s in range(2):
            g = g2 * 2 + s
            wait_in(s)

            @pl.when(g + 1 < NBLK)
            def _():
                start_in(g + 1, 1 - s)

            @pl.when(g >= 2)
            def _():
                wait_out(s)

            src = in_bufs[s]
            dst0 = out_bufs[s].at[pl.ds(0, N)]
            dst1 = out_bufs[s].at[pl.ds(N, N)]

            @plsc.parallel_loop(0, NCHUNK, 1, unroll=UNROLL)
            def chunk(k, src=src, dst0=dst0, dst1=dst1):
                off = k * 16
                pk = idx_v[pl.ds(off, 16)]
                a = lax.shift_right_logical(pk, 8)
                b2 = lax.bitwise_and(pk, 255)
                v0 = src[0, pl.ds(off, 16)]
                v1 = src[1, pl.ds(off, 16)]
                plsc.store_scatter(dst0, [a, b2], v0)
                plsc.store_scatter(dst0, [b2, a], v0)
                plsc.store_scatter(dst1, [a, b2], v1)
                plsc.store_scatter(dst1, [b2, a], v1)

            start_out(g, s)
        return carry

    lax.fori_loop(0, NBLK // 2, outer, 0, unroll=False)
    wait_out(0)
    wait_out(1)


def kernel(input):
    idx = jnp.asarray(_scatter_table())
    out = _unvec_kernel(input, idx)
    return out.reshape(B, N, N)
